# baseline (speedup 1.0000x reference)
import functools

import jax
import jax.numpy as jnp
import numpy as np
from jax.experimental import pallas as pl
from jax.experimental.pallas import tpu as pltpu

_CAP_FACTOR = 1.0
_RB = 512


def _gelu_exact(z):
    return 0.5 * z * (1.0 + jax.lax.erf(z * np.float32(1.0 / np.sqrt(2.0))))


def _router_kernel(x_ref, wg_ref, w1_ref, w2_ref, out_ref,
                   wall_ref, thr_ref, tri_ref, xb_ref, tp_ref, pt_ref,
                   wsel_ref, sel_ref, acc_ref, *, E, C, HB):
    e = pl.program_id(1)
    hb = pl.program_id(2)
    T, D = x_ref.shape

    @pl.when((e == 0) & (hb == 0))
    def _init():
        xb = x_ref[...].astype(jnp.bfloat16)
        xb_ref[...] = xb
        s = jnp.dot(xb, wg_ref[...].astype(jnp.bfloat16),
                    preferred_element_type=jnp.float32)
        m = jnp.max(s, axis=0, keepdims=True)
        ex = jnp.exp(s - m)
        wall = ex / jnp.sum(ex, axis=0, keepdims=True)
        wall_ref[...] = wall
        tp_ref[...] = jnp.zeros_like(tp_ref)
        out_ref[...] = jnp.zeros_like(out_ref)

        lo = jnp.zeros((1, E), jnp.int32)
        hi = jnp.full((1, E), 0x7F800000, jnp.int32)
        for _ in range(31):
            mid = lo + (hi - lo) // 2
            th = jax.lax.bitcast_convert_type(mid, jnp.float32)
            cnt = jnp.sum((wall > th).astype(jnp.float32),
                          axis=0, keepdims=True)
            take = cnt >= C
            lo = jnp.where(take, mid, lo)
            hi = jnp.where(take, hi, mid)
        thr_ref[...] = jax.lax.bitcast_convert_type(hi, jnp.float32)

        tri_ref[...] = (
            jax.lax.broadcasted_iota(jnp.int32, (_RB, _RB), 0)
            > jax.lax.broadcasted_iota(jnp.int32, (_RB, _RB), 1)
        ).astype(jnp.bfloat16)

    @pl.when(hb == 0)
    def _route():
        mrow = jax.lax.broadcasted_iota(jnp.int32, (1, E), 1) == e
        w_col = jnp.sum(jnp.where(mrow, wall_ref[...], 0.0),
                        axis=1, keepdims=True)
        th = jnp.sum(jnp.where(mrow, thr_ref[...], 0.0))

        def _prefix(mask):
            parts = []
            off = jnp.zeros((), jnp.float32)
            for k in range(T // _RB):
                blk = mask[k * _RB:(k + 1) * _RB].astype(jnp.float32)
                p = jnp.dot(tri_ref[...], blk.astype(jnp.bfloat16),
                            preferred_element_type=jnp.float32)
                parts.append(p + off)
                off = off + jnp.sum(blk)
            return jnp.concatenate(parts, axis=0), off

        mask_gt = w_col > th
        mask_eq = w_col == th
        n_gt = jnp.sum(mask_gt.astype(jnp.float32))
        eqp, _ = _prefix(mask_eq)
        sel = mask_gt | (mask_eq & (eqp < (C - n_gt)))
        slot, _ = _prefix(sel)
        wsel_ref[...] = jnp.where(sel, w_col, 0.0)

        iota_c = jax.lax.broadcasted_iota(jnp.int32, (T, C), 1)
        pt = ((slot.astype(jnp.int32) == iota_c) & sel
              ).astype(jnp.bfloat16)
        pt_ref[...] = pt

        sel_ref[...] = jax.lax.dot_general(
            pt, xb_ref[...], (((0,), (0,)), ((), ())),
            preferred_element_type=jnp.float32).astype(jnp.bfloat16)
        acc_ref[...] = jnp.zeros_like(acc_ref)

    z = jnp.dot(sel_ref[...], w1_ref[...],
                preferred_element_type=jnp.float32)
    h = _gelu_exact(z).astype(jnp.bfloat16)
    acc_ref[...] += jnp.dot(h, w2_ref[...],
                            preferred_element_type=jnp.float32)

    @pl.when(hb == HB - 1)
    def _combine():
        wsel = wsel_ref[...]
        contrib = jnp.dot(pt_ref[...], acc_ref[...].astype(jnp.bfloat16),
                          preferred_element_type=jnp.float32)
        out_ref[...] += contrib * wsel
        tp_ref[...] += wsel

    @pl.when((e == E - 1) & (hb == HB - 1))
    def _norm():
        out_ref[...] = out_ref[...] / jnp.maximum(tp_ref[...], 1e-8)


def _forward(x, Wg, W1, W2, interpret=False):
    B, T, D = x.shape
    E = Wg.shape[1]
    H = W1.shape[2]
    C = min(T, max(1, int(T * _CAP_FACTOR / E)))
    HB = 4
    HBLK = H // HB
    w1b = W1.astype(jnp.bfloat16)
    w2b = W2.astype(jnp.bfloat16)
    return pl.pallas_call(
        functools.partial(_router_kernel, E=E, C=C, HB=HB),
        grid=(B, E, HB),
        in_specs=[
            pl.BlockSpec((None, T, D), lambda b, e, hb: (b, 0, 0)),
            pl.BlockSpec((D, E), lambda b, e, hb: (0, 0)),
            pl.BlockSpec((None, D, HBLK), lambda b, e, hb: (e, 0, hb)),
            pl.BlockSpec((None, HBLK, D), lambda b, e, hb: (e, hb, 0)),
        ],
        out_specs=pl.BlockSpec((None, T, D), lambda b, e, hb: (b, 0, 0)),
        out_shape=jax.ShapeDtypeStruct((B, T, D), jnp.float32),
        scratch_shapes=[
            pltpu.VMEM((T, E), jnp.float32),
            pltpu.VMEM((1, E), jnp.float32),
            pltpu.VMEM((_RB, _RB), jnp.bfloat16),
            pltpu.VMEM((T, D), jnp.bfloat16),
            pltpu.VMEM((T, 1), jnp.float32),
            pltpu.VMEM((T, C), jnp.bfloat16),
            pltpu.VMEM((T, 1), jnp.float32),
            pltpu.VMEM((C, D), jnp.bfloat16),
            pltpu.VMEM((C, D), jnp.float32),
        ],
        interpret=interpret,
    )(x, Wg, w1b, w2b)


def kernel(x, Wg, W1, W2):
    return _forward(x, Wg, W1, W2)


# speedup vs baseline: 1.0930x; 1.0930x over previous
import functools

import jax
import jax.numpy as jnp
import numpy as np
from jax.experimental import pallas as pl
from jax.experimental.pallas import tpu as pltpu

_CAP_FACTOR = 1.0
_RB = 512


def _gelu_exact(z):
    return 0.5 * z * (1.0 + jax.lax.erf(z * np.float32(1.0 / np.sqrt(2.0))))


def _router_kernel(xb_ref, wg_ref, w1_ref, w2_ref, out_ref,
                   wall_ref, thr_ref, tri_ref, tp_ref, pt_ref,
                   wsel_ref, sel_ref, acc_ref, *, B, E, C, HB):
    e = pl.program_id(0)
    hb = pl.program_id(1)
    _, T, D = xb_ref.shape

    @pl.when((e == 0) & (hb == 0))
    def _init():
        x2 = xb_ref[...].reshape(B * T, D)
        s = jnp.dot(x2, wg_ref[...],
                    preferred_element_type=jnp.float32)
        walls = []
        for b in range(B):
            sb = s[b * T:(b + 1) * T]
            m = jnp.max(sb, axis=0, keepdims=True)
            ex = jnp.exp(sb - m)
            wb = ex / jnp.sum(ex, axis=0, keepdims=True)
            walls.append(wb)
            lo = jnp.zeros((1, E), jnp.int32)
            hi = jnp.full((1, E), 0x7F800000, jnp.int32)
            for _ in range(31):
                mid = lo + (hi - lo) // 2
                th = jax.lax.bitcast_convert_type(mid, jnp.float32)
                cnt = jnp.sum((wb > th).astype(jnp.float32),
                              axis=0, keepdims=True)
                take = cnt >= C
                lo = jnp.where(take, mid, lo)
                hi = jnp.where(take, hi, mid)
            thr_ref[b:b + 1, :] = jax.lax.bitcast_convert_type(
                hi, jnp.float32)
        wall_ref[...] = jnp.concatenate(walls, axis=0)
        tp_ref[...] = jnp.zeros_like(tp_ref)
        out_ref[...] = jnp.zeros_like(out_ref)

        tri_ref[...] = (
            jax.lax.broadcasted_iota(jnp.int32, (_RB, _RB), 0)
            > jax.lax.broadcasted_iota(jnp.int32, (_RB, _RB), 1)
        ).astype(jnp.bfloat16)

    mrow = jax.lax.broadcasted_iota(jnp.int32, (1, E), 1) == e

    @pl.when(hb == 0)
    def _route():
        for b in range(B):
            w_col = jnp.sum(
                jnp.where(mrow, wall_ref[b * T:(b + 1) * T, :], 0.0),
                axis=1, keepdims=True)
            th = jnp.sum(jnp.where(mrow, thr_ref[b:b + 1, :], 0.0))

            def _prefix(mask):
                parts = []
                off = jnp.zeros((), jnp.float32)
                for k in range(T // _RB):
                    blk = mask[k * _RB:(k + 1) * _RB].astype(jnp.float32)
                    p = jnp.dot(tri_ref[...], blk.astype(jnp.bfloat16),
                                preferred_element_type=jnp.float32)
                    parts.append(p + off)
                    off = off + jnp.sum(blk)
                return jnp.concatenate(parts, axis=0)

            mask_gt = w_col > th
            mask_eq = w_col == th
            n_gt = jnp.sum(mask_gt.astype(jnp.float32))
            eqp = _prefix(mask_eq)
            sel = mask_gt | (mask_eq & (eqp < (C - n_gt)))
            slot = _prefix(sel)
            wsel_ref[:, b:b + 1] = jnp.where(sel, w_col, 0.0)

            iota_c = jax.lax.broadcasted_iota(jnp.int32, (T, C), 1)
            pt = ((slot.astype(jnp.int32) == iota_c) & sel
                  ).astype(jnp.bfloat16)
            pt_ref[b] = pt

            sel_ref[b * C:(b + 1) * C, :] = jax.lax.dot_general(
                pt, xb_ref[b], (((0,), (0,)), ((), ())),
                preferred_element_type=jnp.float32).astype(jnp.bfloat16)
        acc_ref[...] = jnp.zeros_like(acc_ref)

    z = jnp.dot(sel_ref[...], w1_ref[...],
                preferred_element_type=jnp.float32)
    h = _gelu_exact(z).astype(jnp.bfloat16)
    acc_ref[...] += jnp.dot(h, w2_ref[...],
                            preferred_element_type=jnp.float32)

    @pl.when(hb == HB - 1)
    def _combine():
        for b in range(B):
            wsel = wsel_ref[:, b:b + 1]
            contrib = jnp.dot(
                pt_ref[b],
                acc_ref[b * C:(b + 1) * C, :].astype(jnp.bfloat16),
                preferred_element_type=jnp.float32)
            out_ref[b] += contrib * wsel
            tp_ref[:, b:b + 1] += wsel

    @pl.when((e == E - 1) & (hb == HB - 1))
    def _norm():
        for b in range(B):
            out_ref[b] = out_ref[b] / jnp.maximum(
                tp_ref[:, b:b + 1], 1e-8)


def _forward(x, Wg, W1, W2, interpret=False):
    B, T, D = x.shape
    E = Wg.shape[1]
    H = W1.shape[2]
    C = min(T, max(1, int(T * _CAP_FACTOR / E)))
    HB = 4
    HBLK = H // HB
    xb = x.astype(jnp.bfloat16)
    wgb = Wg.astype(jnp.bfloat16)
    w1b = W1.astype(jnp.bfloat16)
    w2b = W2.astype(jnp.bfloat16)
    return pl.pallas_call(
        functools.partial(_router_kernel, B=B, E=E, C=C, HB=HB),
        grid=(E, HB),
        in_specs=[
            pl.BlockSpec((B, T, D), lambda e, hb: (0, 0, 0)),
            pl.BlockSpec((D, E), lambda e, hb: (0, 0)),
            pl.BlockSpec((None, D, HBLK), lambda e, hb: (e, 0, hb)),
            pl.BlockSpec((None, HBLK, D), lambda e, hb: (e, hb, 0)),
        ],
        out_specs=pl.BlockSpec((B, T, D), lambda e, hb: (0, 0, 0)),
        out_shape=jax.ShapeDtypeStruct((B, T, D), jnp.float32),
        scratch_shapes=[
            pltpu.VMEM((B * T, E), jnp.float32),
            pltpu.VMEM((B, E), jnp.float32),
            pltpu.VMEM((_RB, _RB), jnp.bfloat16),
            pltpu.VMEM((T, B), jnp.float32),
            pltpu.VMEM((B, T, C), jnp.bfloat16),
            pltpu.VMEM((T, B), jnp.float32),
            pltpu.VMEM((B * C, D), jnp.bfloat16),
            pltpu.VMEM((B * C, D), jnp.float32),
        ],
        compiler_params=pltpu.CompilerParams(
            vmem_limit_bytes=64 * 1024 * 1024),
        interpret=interpret,
    )(xb, wgb, w1b, w2b)


def kernel(x, Wg, W1, W2):
    return _forward(x, Wg, W1, W2)


# speedup vs baseline: 1.1049x; 1.0109x over previous
import functools

import jax
import jax.numpy as jnp
import numpy as np
from jax.experimental import pallas as pl
from jax.experimental.pallas import tpu as pltpu

_CAP_FACTOR = 1.0
_RB = 512


def _gelu_exact(z):
    return 0.5 * z * (1.0 + jax.lax.erf(z * np.float32(1.0 / np.sqrt(2.0))))


def _router_kernel(xb_ref, wg_ref, w1_ref, w2_ref, out_ref,
                   wall_ref, thr_ref, tri_ref, tp_ref, pt_ref,
                   wsel_ref, sel_ref, acc_ref, *, B, E, C, HB):
    e = pl.program_id(0)
    hb = pl.program_id(1)
    _, T, D = xb_ref.shape

    @pl.when((e == 0) & (hb == 0))
    def _init():
        x2 = xb_ref[...].reshape(B * T, D)
        s = jnp.dot(x2, wg_ref[...],
                    preferred_element_type=jnp.float32)
        walls = []
        for b in range(B):
            sb = s[b * T:(b + 1) * T]
            m = jnp.max(sb, axis=0, keepdims=True)
            ex = jnp.exp(sb - m)
            wb = ex / jnp.sum(ex, axis=0, keepdims=True)
            walls.append(wb)
            lo = jnp.zeros((1, E), jnp.int32)
            hi = jnp.full((1, E), 0x7F800000, jnp.int32)
            for _ in range(31):
                mid = lo + (hi - lo) // 2
                th = jax.lax.bitcast_convert_type(mid, jnp.float32)
                cnt = jnp.sum((wb > th).astype(jnp.float32),
                              axis=0, keepdims=True)
                take = cnt >= C
                lo = jnp.where(take, mid, lo)
                hi = jnp.where(take, hi, mid)
            thr_ref[b:b + 1, :] = jax.lax.bitcast_convert_type(
                hi, jnp.float32)
        wall_ref[...] = jnp.concatenate(walls, axis=0)
        tp_ref[...] = jnp.zeros_like(tp_ref)
        out_ref[...] = jnp.zeros_like(out_ref)

        tri_ref[...] = (
            jax.lax.broadcasted_iota(jnp.int32, (_RB, _RB), 0)
            > jax.lax.broadcasted_iota(jnp.int32, (_RB, _RB), 1)
        ).astype(jnp.bfloat16)

    mrow = jax.lax.broadcasted_iota(jnp.int32, (1, E), 1) == e

    @pl.when(hb == 0)
    def _route():
        for b in range(B):
            w_col = jnp.sum(
                jnp.where(mrow, wall_ref[b * T:(b + 1) * T, :], 0.0),
                axis=1, keepdims=True)
            th = jnp.sum(jnp.where(mrow, thr_ref[b:b + 1, :], 0.0))

            def _prefix(mask):
                parts = []
                off = jnp.zeros((), jnp.float32)
                for k in range(T // _RB):
                    blk = mask[k * _RB:(k + 1) * _RB].astype(jnp.float32)
                    p = jnp.dot(tri_ref[...], blk.astype(jnp.bfloat16),
                                preferred_element_type=jnp.float32)
                    parts.append(p + off)
                    off = off + jnp.sum(blk)
                return jnp.concatenate(parts, axis=0)

            mask_gt = w_col > th
            mask_eq = w_col == th
            n_gt = jnp.sum(mask_gt.astype(jnp.float32))
            eqp = _prefix(mask_eq)
            sel = mask_gt | (mask_eq & (eqp < (C - n_gt)))
            slot = _prefix(sel)
            wsel_ref[:, b:b + 1] = jnp.where(sel, w_col, 0.0)

            iota_c = jax.lax.broadcasted_iota(jnp.int32, (T, C), 1)
            pt = ((slot.astype(jnp.int32) == iota_c) & sel
                  ).astype(jnp.bfloat16)
            pt_ref[b] = pt

            sel_ref[b * C:(b + 1) * C, :] = jax.lax.dot_general(
                pt, xb_ref[b], (((0,), (0,)), ((), ())),
                preferred_element_type=jnp.float32).astype(jnp.bfloat16)
        acc_ref[...] = jnp.zeros_like(acc_ref)

    z = jnp.dot(sel_ref[...], w1_ref[...],
                preferred_element_type=jnp.float32)
    h = _gelu_exact(z).astype(jnp.bfloat16)
    acc_ref[...] += jnp.dot(h, w2_ref[...],
                            preferred_element_type=jnp.float32)

    @pl.when(hb == HB - 1)
    def _combine():
        for b in range(B):
            wsel = wsel_ref[:, b:b + 1]
            contrib = jnp.dot(
                pt_ref[b],
                acc_ref[b * C:(b + 1) * C, :].astype(jnp.bfloat16),
                preferred_element_type=jnp.float32)
            out_ref[b] += contrib * wsel
            tp_ref[:, b:b + 1] += wsel

    @pl.when((e == E - 1) & (hb == HB - 1))
    def _norm():
        for b in range(B):
            out_ref[b] = out_ref[b] / jnp.maximum(
                tp_ref[:, b:b + 1], 1e-8)


def _forward(x, Wg, W1, W2, interpret=False):
    B, T, D = x.shape
    E = Wg.shape[1]
    H = W1.shape[2]
    C = min(T, max(1, int(T * _CAP_FACTOR / E)))
    HB = 2
    HBLK = H // HB
    xb = x.astype(jnp.bfloat16)
    wgb = Wg.astype(jnp.bfloat16)
    w1b = W1.astype(jnp.bfloat16)
    w2b = W2.astype(jnp.bfloat16)
    return pl.pallas_call(
        functools.partial(_router_kernel, B=B, E=E, C=C, HB=HB),
        grid=(E, HB),
        in_specs=[
            pl.BlockSpec((B, T, D), lambda e, hb: (0, 0, 0)),
            pl.BlockSpec((D, E), lambda e, hb: (0, 0)),
            pl.BlockSpec((None, D, HBLK), lambda e, hb: (e, 0, hb)),
            pl.BlockSpec((None, HBLK, D), lambda e, hb: (e, hb, 0)),
        ],
        out_specs=pl.BlockSpec((B, T, D), lambda e, hb: (0, 0, 0)),
        out_shape=jax.ShapeDtypeStruct((B, T, D), jnp.float32),
        scratch_shapes=[
            pltpu.VMEM((B * T, E), jnp.float32),
            pltpu.VMEM((B, E), jnp.float32),
            pltpu.VMEM((_RB, _RB), jnp.bfloat16),
            pltpu.VMEM((T, B), jnp.float32),
            pltpu.VMEM((B, T, C), jnp.bfloat16),
            pltpu.VMEM((T, B), jnp.float32),
            pltpu.VMEM((B * C, D), jnp.bfloat16),
            pltpu.VMEM((B * C, D), jnp.float32),
        ],
        compiler_params=pltpu.CompilerParams(
            vmem_limit_bytes=64 * 1024 * 1024),
        interpret=interpret,
    )(xb, wgb, w1b, w2b)


def kernel(x, Wg, W1, W2):
    return _forward(x, Wg, W1, W2)
